# baseline (speedup 1.0000x reference)
import functools

import jax
import jax.numpy as jnp
from jax import lax
from jax.experimental import pallas as pl
from jax.experimental.pallas import tpu as pltpu
from jax.experimental.pallas import tpu_sc as plsc

B = 16
MAXLEN = 512
SEQ = 128
CLS = 101
SEP = 102
PAD = 0
BUDGET = SEQ - 3
H1 = (BUDGET + 1) // 2
WIN = SEQ
NCORES = 1
HALF = SEQ // NCORES
NVREG = HALF // 16

_MESH = plsc.VectorSubcoreMesh(
    core_axis_name="c", subcore_axis_name="s", num_cores=NCORES,
    num_subcores=16
)


def _body(t1_hbm, l1_hbm, t2_hbm, l2_hbm,
          ow_hbm, om_hbm, ot_hbm,
          l1_v, l2_v, t1_v, t2_v, ow_v, om_v, ot_v,
          sem_len, sem_tok, sem_out):
    c = lax.axis_index("c")
    row = lax.axis_index("s")
    base = c * HALF

    lc1 = pltpu.async_copy(l1_hbm, l1_v, sem_len)
    lc2 = pltpu.async_copy(l2_hbm, l2_v, sem_len)
    tc1 = pltpu.async_copy(t1_hbm.at[row, pl.ds(0, WIN)], t1_v, sem_tok)
    tc2 = pltpu.async_copy(t2_hbm.at[row, pl.ds(0, WIN)], t2_v, sem_tok)

    lc1.wait()
    lc2.wait()
    rowv = jnp.full((16,), row, dtype=jnp.int32)
    l1b = plsc.load_gather(l1_v, [rowv])
    l2b = plsc.load_gather(l2_v, [rowv])
    k1 = jnp.minimum(l1b, jnp.maximum(H1, BUDGET - l2b))
    k2 = jnp.minimum(l2b, BUDGET - k1)
    sep1_at = 1 + k1
    sep2_at = 2 + k1 + k2

    lane = lax.iota(jnp.int32, 16)
    for j in range(NVREG):
        pos = lane + (base + j * 16)
        sl = pl.ds(j * 16, 16)
        om_v[sl] = (pos <= sep2_at).astype(jnp.int32)
        ot_v[sl] = ((pos > sep1_at) & (pos <= sep2_at)).astype(jnp.int32)
    oc2 = pltpu.async_copy(om_v, om_hbm.at[row, pl.ds(base, HALF)], sem_out)
    oc3 = pltpu.async_copy(ot_v, ot_hbm.at[row, pl.ds(base, HALF)], sem_out)

    tc1.wait()
    tc2.wait()
    ow_copies = []
    for j in range(NVREG):
        pos = lane + (base + j * 16)
        if j == 0:
            t1 = plsc.load_gather(t1_v, [jnp.maximum(pos - 1, 0)])
        else:
            t1 = t1_v[pl.ds(base + j * 16 - 1, 16)]
        idx2 = jnp.clip(pos - 2 - k1, 0, WIN - 1)
        t2 = plsc.load_gather(t2_v, [idx2])
        in_seg1 = (pos >= 1) & (pos < sep1_at)
        in_seg2 = (pos > sep1_at) & (pos < sep2_at)
        wid = jnp.where(
            pos == 0, CLS,
            jnp.where(in_seg1, t1,
                      jnp.where(pos == sep1_at, SEP,
                                jnp.where(in_seg2, t2,
                                          jnp.where(pos == sep2_at, SEP, PAD)))))
        ow_v[pl.ds(j * 16, 16)] = wid.astype(jnp.int32)
        if (j + 1) % (NVREG // 2) == 0:
            half0 = (j + 1 - NVREG // 2) * 16
            ow_copies.append(pltpu.async_copy(
                ow_v.at[pl.ds(half0, HALF // 2)],
                ow_hbm.at[row, pl.ds(base + half0, HALF // 2)], sem_out))

    for cpy in ow_copies:
        cpy.wait()
    oc2.wait()
    oc3.wait()


_packer = functools.partial(
    pl.kernel,
    out_type=(
        jax.ShapeDtypeStruct((B, SEQ), jnp.int32),
        jax.ShapeDtypeStruct((B, SEQ), jnp.int32),
        jax.ShapeDtypeStruct((B, SEQ), jnp.int32),
    ),
    mesh=_MESH,
    compiler_params=pltpu.CompilerParams(
        needs_layout_passes=False,
        disable_bounds_checks=True,
        disable_semaphore_checks=True,
        skip_device_barrier=True,
    ),
    scratch_types=[
        pltpu.VMEM((16,), jnp.int32),
        pltpu.VMEM((16,), jnp.int32),
        pltpu.VMEM((WIN,), jnp.int32),
        pltpu.VMEM((WIN,), jnp.int32),
        pltpu.VMEM((HALF,), jnp.int32),
        pltpu.VMEM((HALF,), jnp.int32),
        pltpu.VMEM((HALF,), jnp.int32),
        pltpu.SemaphoreType.DMA,
        pltpu.SemaphoreType.DMA,
        pltpu.SemaphoreType.DMA,
    ],
)(_body)


def kernel(tokens1, len1, tokens2, len2):
    return _packer(tokens1, len1.astype(jnp.int32), tokens2,
                   len2.astype(jnp.int32))


# speedup vs baseline: 1.0123x; 1.0123x over previous
import functools

import jax
import jax.numpy as jnp
from jax import lax
from jax.experimental import pallas as pl
from jax.experimental.pallas import tpu as pltpu
from jax.experimental.pallas import tpu_sc as plsc

B = 16
MAXLEN = 512
SEQ = 128
CLS = 101
SEP = 102
PAD = 0
BUDGET = SEQ - 3
H1 = (BUDGET + 1) // 2
WIN = SEQ
NCORES = 1
HALF = SEQ // NCORES
NVREG = HALF // 16

_MESH = plsc.VectorSubcoreMesh(
    core_axis_name="c", subcore_axis_name="s", num_cores=NCORES,
    num_subcores=16
)


def _body(t1_hbm, l1_hbm, t2_hbm, l2_hbm,
          ow_hbm, om_hbm, ot_hbm,
          l1_v, l2_v, t1_v, t2_v, ow_v, om_v, ot_v,
          sem_len, sem_tok, sem_out):
    c = lax.axis_index("c")
    row = lax.axis_index("s")
    base = c * HALF

    tc1 = pltpu.async_copy(t1_hbm.at[row, pl.ds(0, WIN)], t1_v, sem_tok)
    tc2 = pltpu.async_copy(t2_hbm.at[row, pl.ds(0, WIN)], t2_v, sem_tok)
    lc1 = pltpu.async_copy(l1_hbm, l1_v, sem_len)
    lc2 = pltpu.async_copy(l2_hbm, l2_v, sem_len)

    lc1.wait()
    lc2.wait()
    rowv = jnp.full((16,), row, dtype=jnp.int32)
    l1b = plsc.load_gather(l1_v, [rowv])
    l2b = plsc.load_gather(l2_v, [rowv])
    k1 = jnp.minimum(l1b, jnp.maximum(H1, BUDGET - l2b))
    k2 = jnp.minimum(l2b, BUDGET - k1)
    sep1_at = 1 + k1
    sep2_at = 2 + k1 + k2

    lane = lax.iota(jnp.int32, 16)
    for j in range(NVREG):
        pos = lane + (base + j * 16)
        sl = pl.ds(j * 16, 16)
        om_v[sl] = (pos <= sep2_at).astype(jnp.int32)
        ot_v[sl] = ((pos > sep1_at) & (pos <= sep2_at)).astype(jnp.int32)
    oc2 = pltpu.async_copy(om_v, om_hbm.at[row, pl.ds(base, HALF)], sem_out)
    oc3 = pltpu.async_copy(ot_v, ot_hbm.at[row, pl.ds(base, HALF)], sem_out)

    tc1.wait()
    tc2.wait()
    for j in range(NVREG):
        pos = lane + (base + j * 16)
        if j == 0:
            t1 = plsc.load_gather(t1_v, [jnp.maximum(pos - 1, 0)])
        else:
            t1 = t1_v[pl.ds(base + j * 16 - 1, 16)]
        idx2 = jnp.clip(pos - 2 - k1, 0, WIN - 1)
        t2 = plsc.load_gather(t2_v, [idx2])
        in_seg1 = (pos >= 1) & (pos < sep1_at)
        in_seg2 = (pos > sep1_at) & (pos < sep2_at)
        wid = jnp.where(
            pos == 0, CLS,
            jnp.where(in_seg1, t1,
                      jnp.where(pos == sep1_at, SEP,
                                jnp.where(in_seg2, t2,
                                          jnp.where(pos == sep2_at, SEP, PAD)))))
        ow_v[pl.ds(j * 16, 16)] = wid.astype(jnp.int32)
    oc1 = pltpu.async_copy(ow_v, ow_hbm.at[row, pl.ds(base, HALF)], sem_out)

    oc1.wait()
    oc2.wait()
    oc3.wait()


_packer = functools.partial(
    pl.kernel,
    out_type=(
        jax.ShapeDtypeStruct((B, SEQ), jnp.int32),
        jax.ShapeDtypeStruct((B, SEQ), jnp.int32),
        jax.ShapeDtypeStruct((B, SEQ), jnp.int32),
    ),
    mesh=_MESH,
    compiler_params=pltpu.CompilerParams(
        needs_layout_passes=False,
        disable_bounds_checks=True,
        disable_semaphore_checks=True,
    ),
    scratch_types=[
        pltpu.VMEM((16,), jnp.int32),
        pltpu.VMEM((16,), jnp.int32),
        pltpu.VMEM((WIN,), jnp.int32),
        pltpu.VMEM((WIN,), jnp.int32),
        pltpu.VMEM((HALF,), jnp.int32),
        pltpu.VMEM((HALF,), jnp.int32),
        pltpu.VMEM((HALF,), jnp.int32),
        pltpu.SemaphoreType.DMA,
        pltpu.SemaphoreType.DMA,
        pltpu.SemaphoreType.DMA,
    ],
)(_body)


def kernel(tokens1, len1, tokens2, len2):
    return _packer(tokens1, len1.astype(jnp.int32), tokens2,
                   len2.astype(jnp.int32))
